# baseline (speedup 1.0000x reference)
import functools
import math

import jax
import jax.numpy as jnp
from jax import lax
from jax.experimental import pallas as pl
from jax.experimental.pallas import tpu as pltpu
from jax.experimental.pallas import tpu_sc as plsc

N_CORES = 2
N_SUBCORES = 16
N_WORKERS = N_CORES * N_SUBCORES
CHUNK = 128
LANES = 16

F_BLK = 2048


def _transpose_in_kernel(x_ref, eye_ref, o_ref):
    o_ref[...] = lax.dot_general(
        x_ref[...], eye_ref[...], (((0,), (0,)), ((), ())),
        preferred_element_type=jnp.float32)


def _epilogue_kernel(p_ref, bias_ref, eye_ref, o_ref):
    s = p_ref[0] + p_ref[1]
    t = lax.dot_general(eye_ref[...], s, (((1,), (1,)), ((), ())),
                        preferred_element_type=jnp.float32)
    o_ref[...] = t + bias_ref[...]


def _make_sc_kernel(n_features, batch, cpw):
    mesh = plsc.VectorSubcoreMesh(core_axis_name="c", subcore_axis_name="s")
    rows_per_tile = n_features // N_SUBCORES
    stage_iters = rows_per_tile // CHUNK

    @functools.partial(
        pl.kernel,
        out_type=jax.ShapeDtypeStruct((N_CORES, n_features, batch),
                                      jnp.float32),
        mesh=mesh,
        scratch_types=[
            pltpu.VMEM_SHARED((n_features, batch), jnp.float32),
            pltpu.VMEM((cpw, CHUNK), jnp.float32),
            pltpu.VMEM((cpw, CHUNK), jnp.int32),
            pltpu.VMEM((cpw, CHUNK), jnp.int32),
            [pltpu.VMEM((CHUNK, batch), jnp.float32)] * 4,
            [pltpu.SemaphoreType.DMA] * 4,
            [pltpu.SemaphoreType.DMA] * 4,
        ],
        compiler_params=pltpu.CompilerParams(use_tc_tiling_on_sc=False),
    )
    def sc_kernel(tableT, vals_h, rows_h, cols_h, out, acc, vals_v, rows_v,
                  cols_v, bufs, gsems, ssems):
        cid = lax.axis_index("c")
        sid = lax.axis_index("s")
        wid = cid * N_SUBCORES + sid

        pltpu.sync_copy(vals_h.at[wid], vals_v)
        pltpu.sync_copy(rows_h.at[wid], rows_v)
        pltpu.sync_copy(cols_h.at[wid], cols_v)

        zero16 = jnp.zeros((LANES,), jnp.float32)

        def zero_body(r, carry):
            for jf in range(batch // LANES):
                bufs[0][r, pl.ds(jf * LANES, LANES)] = zero16
            return carry

        lax.fori_loop(0, CHUNK, zero_body, 0)

        def zcp_body(i, carry):
            pltpu.sync_copy(
                bufs[0],
                acc.at[pl.ds(sid * rows_per_tile + i * CHUNK, CHUNK)])
            return carry

        lax.fori_loop(0, stage_iters, zcp_body, 0)
        plsc.subcore_barrier()

        def scale_buf(buf, j):
            def mul_body(g, c2):
                vv16 = vals_v[j, pl.ds(g * LANES, LANES)]
                for k in range(LANES):
                    vv = jnp.full((LANES,), vv16[k], jnp.float32)
                    e = g * LANES + k
                    for jf in range(batch // LANES):
                        sl = pl.ds(jf * LANES, LANES)
                        buf[e, sl] = buf[e, sl] * vv
                return c2

            lax.fori_loop(0, CHUNK // LANES, mul_body, 0)

        n_iter = cpw // 4

        for k in range(3):
            pltpu.async_copy(tableT.at[cols_v.at[k]], bufs[k], gsems[k])

        def ring_body(i, carry):
            for k in range(4):
                j = 4 * i + k
                b = bufs[k]
                pltpu.make_async_copy(
                    tableT.at[cols_v.at[j]], b, gsems[k]).wait()
                scale_buf(b, j)
                pltpu.async_copy(b, acc.at[rows_v.at[j]], ssems[k],
                                 add=True)
                kp = (k + 3) % 4
                if k == 0:
                    @pl.when(i >= 1)
                    def _():
                        pltpu.make_async_copy(
                            bufs[kp], acc.at[rows_v.at[j - 1]],
                            ssems[kp]).wait()

                    pltpu.async_copy(tableT.at[cols_v.at[j + 3]],
                                     bufs[kp], gsems[kp])
                else:
                    @pl.when(j + 3 < cpw)
                    def _():
                        pltpu.make_async_copy(
                            bufs[kp], acc.at[rows_v.at[j - 1]],
                            ssems[kp]).wait()
                        pltpu.async_copy(tableT.at[cols_v.at[j + 3]],
                                         bufs[kp], gsems[kp])
            return carry

        lax.fori_loop(0, n_iter, ring_body, 0)

        for k in range(4):
            pltpu.make_async_copy(
                bufs[k], acc.at[rows_v.at[cpw - 4 + k]], ssems[k]).wait()
        plsc.subcore_barrier()

        def wb_body(i, carry):
            r0 = sid * rows_per_tile + i * CHUNK
            pltpu.sync_copy(acc.at[pl.ds(r0, CHUNK)], bufs[0])
            pltpu.sync_copy(bufs[0], out.at[cid, pl.ds(r0, CHUNK)])
            return carry

        lax.fori_loop(0, stage_iters, wb_body, 0)

    return sc_kernel


def kernel(inputs, kernel_vals, kernel_rows, kernel_cols, bias):
    batch, in_dim = inputs.shape
    n_features = bias.shape[0]
    nnz = kernel_vals.shape[0]

    cpw = math.ceil(nnz / (N_WORKERS * CHUNK))
    cpw = -(-cpw // 4) * 4
    nnzp = N_WORKERS * cpw * CHUNK
    pad = nnzp - nnz
    vals_p = jnp.concatenate(
        [kernel_vals, jnp.zeros((pad,), jnp.float32)]).reshape(
            N_WORKERS, cpw, CHUNK)
    rows_p = jnp.concatenate(
        [kernel_rows.astype(jnp.int32), jnp.zeros((pad,), jnp.int32)]
    ).reshape(N_WORKERS, cpw, CHUNK)
    cols_p = jnp.concatenate(
        [kernel_cols.astype(jnp.int32), jnp.zeros((pad,), jnp.int32)]
    ).reshape(N_WORKERS, cpw, CHUNK)

    eye = jnp.eye(batch, dtype=jnp.float32)

    n_blk = in_dim // F_BLK
    inputsT = pl.pallas_call(
        _transpose_in_kernel,
        grid=(n_blk,),
        in_specs=[
            pl.BlockSpec((batch, F_BLK), lambda i: (0, i)),
            pl.BlockSpec((batch, batch), lambda i: (0, 0)),
        ],
        out_specs=pl.BlockSpec((F_BLK, batch), lambda i: (i, 0)),
        out_shape=jax.ShapeDtypeStruct((in_dim, batch), jnp.float32),
    )(inputs, eye)

    parts = _make_sc_kernel(n_features, batch, cpw)(
        inputsT, vals_p, rows_p, cols_p)

    f_blk_n = n_features // F_BLK
    z = pl.pallas_call(
        _epilogue_kernel,
        grid=(f_blk_n,),
        in_specs=[
            pl.BlockSpec((N_CORES, F_BLK, batch), lambda i: (0, i, 0)),
            pl.BlockSpec((1, F_BLK), lambda i: (0, i)),
            pl.BlockSpec((batch, batch), lambda i: (0, 0)),
        ],
        out_specs=pl.BlockSpec((batch, F_BLK), lambda i: (0, i)),
        out_shape=jax.ShapeDtypeStruct((batch, n_features), jnp.float32),
    )(parts, bias.reshape(1, n_features), eye)

    return z


# speedup vs baseline: 2.1655x; 2.1655x over previous
import functools
import math

import jax
import jax.numpy as jnp
from jax import lax
from jax.experimental import pallas as pl
from jax.experimental.pallas import tpu as pltpu
from jax.experimental.pallas import tpu_sc as plsc

N_CORES = 2
N_SUBCORES = 16
N_WORKERS = N_CORES * N_SUBCORES
CHUNK = 128
LANES = 16

F_BLK = 2048


def _transpose_in_kernel(x_ref, eye_ref, o_ref):
    o_ref[...] = lax.dot_general(
        x_ref[...], eye_ref[...], (((0,), (0,)), ((), ())),
        preferred_element_type=jnp.float32)


def _epilogue_kernel(p_ref, bias_ref, eye_ref, o_ref):
    s = p_ref[0] + p_ref[1]
    t = lax.dot_general(eye_ref[...], s, (((1,), (1,)), ((), ())),
                        preferred_element_type=jnp.float32)
    o_ref[...] = t + bias_ref[...]


def _make_sc_kernel(n_features, batch, cpw):
    mesh = plsc.VectorSubcoreMesh(core_axis_name="c", subcore_axis_name="s")
    rows_per_tile = n_features // N_SUBCORES
    stage_iters = rows_per_tile // CHUNK

    @functools.partial(
        pl.kernel,
        out_type=jax.ShapeDtypeStruct((N_CORES, n_features, batch),
                                      jnp.float32),
        mesh=mesh,
        scratch_types=[
            pltpu.VMEM_SHARED((n_features, batch), jnp.float32),
            pltpu.VMEM((cpw, CHUNK), jnp.float32),
            pltpu.VMEM((cpw, CHUNK), jnp.int32),
            pltpu.VMEM((cpw, CHUNK), jnp.int32),
            [pltpu.VMEM((CHUNK, batch), jnp.float32)] * 2,
            [pltpu.SemaphoreType.DMA] * 2,
        ],
        compiler_params=pltpu.CompilerParams(use_tc_tiling_on_sc=False),
    )
    def sc_kernel(tableT, vals_h, rows_h, cols_h, out, acc, vals_v, rows_v,
                  cols_v, bufs, gsems):
        cid = lax.axis_index("c")
        sid = lax.axis_index("s")
        wid = cid * N_SUBCORES + sid

        pltpu.sync_copy(vals_h.at[wid], vals_v)
        pltpu.sync_copy(rows_h.at[wid], rows_v)
        pltpu.sync_copy(cols_h.at[wid], cols_v)

        zero16 = jnp.zeros((LANES,), jnp.float32)

        def zero_body(r, carry):
            for jf in range(batch // LANES):
                bufs[0][r, pl.ds(jf * LANES, LANES)] = zero16
            return carry

        lax.fori_loop(0, CHUNK, zero_body, 0)

        def zcp_body(i, carry):
            pltpu.sync_copy(
                bufs[0],
                acc.at[pl.ds(sid * rows_per_tile + i * CHUNK, CHUNK)])
            return carry

        lax.fori_loop(0, stage_iters, zcp_body, 0)
        plsc.subcore_barrier()

        def scale_buf(buf, j):
            @functools.partial(plsc.parallel_loop, 0, CHUNK // LANES,
                               unroll=4)
            def _(g):
                vv16 = vals_v[j, pl.ds(g * LANES, LANES)]
                for k in range(LANES):
                    vv = jnp.full((LANES,), vv16[k], jnp.float32)
                    e = g * LANES + k
                    for jf in range(batch // LANES):
                        sl = pl.ds(jf * LANES, LANES)
                        buf[e, sl] = buf[e, sl] * vv

        half = cpw // 2

        pltpu.async_copy(tableT.at[cols_v.at[0]], bufs[0], gsems[0])

        def chunk_body(i, carry):
            j0 = 2 * i
            j1 = j0 + 1
            pltpu.async_copy(tableT.at[cols_v.at[j1]], bufs[1], gsems[1])
            pltpu.make_async_copy(
                tableT.at[cols_v.at[j0]], bufs[0], gsems[0]).wait()
            scale_buf(bufs[0], j0)
            pltpu.sync_copy(bufs[0], acc.at[rows_v.at[j0]], add=True)

            @pl.when(i + 1 < half)
            def _():
                pltpu.async_copy(tableT.at[cols_v.at[j0 + 2]], bufs[0],
                                 gsems[0])

            pltpu.make_async_copy(
                tableT.at[cols_v.at[j1]], bufs[1], gsems[1]).wait()
            scale_buf(bufs[1], j1)
            pltpu.sync_copy(bufs[1], acc.at[rows_v.at[j1]], add=True)
            return carry

        lax.fori_loop(0, half, chunk_body, 0)
        plsc.subcore_barrier()

        def wb_body(i, carry):
            r0 = sid * rows_per_tile + i * CHUNK
            pltpu.sync_copy(acc.at[pl.ds(r0, CHUNK)], bufs[0])
            pltpu.sync_copy(bufs[0], out.at[cid, pl.ds(r0, CHUNK)])
            return carry

        lax.fori_loop(0, stage_iters, wb_body, 0)

    return sc_kernel


def kernel(inputs, kernel_vals, kernel_rows, kernel_cols, bias):
    batch, in_dim = inputs.shape
    n_features = bias.shape[0]
    nnz = kernel_vals.shape[0]

    cpw = math.ceil(nnz / (N_WORKERS * CHUNK))
    cpw += cpw % 2
    nnzp = N_WORKERS * cpw * CHUNK
    pad = nnzp - nnz
    vals_p = jnp.concatenate(
        [kernel_vals, jnp.zeros((pad,), jnp.float32)]).reshape(
            N_WORKERS, cpw, CHUNK)
    rows_p = jnp.concatenate(
        [kernel_rows.astype(jnp.int32), jnp.zeros((pad,), jnp.int32)]
    ).reshape(N_WORKERS, cpw, CHUNK)
    cols_p = jnp.concatenate(
        [kernel_cols.astype(jnp.int32), jnp.zeros((pad,), jnp.int32)]
    ).reshape(N_WORKERS, cpw, CHUNK)

    eye = jnp.eye(batch, dtype=jnp.float32)

    n_blk = in_dim // F_BLK
    inputsT = pl.pallas_call(
        _transpose_in_kernel,
        grid=(n_blk,),
        in_specs=[
            pl.BlockSpec((batch, F_BLK), lambda i: (0, i)),
            pl.BlockSpec((batch, batch), lambda i: (0, 0)),
        ],
        out_specs=pl.BlockSpec((F_BLK, batch), lambda i: (i, 0)),
        out_shape=jax.ShapeDtypeStruct((in_dim, batch), jnp.float32),
    )(inputs, eye)

    parts = _make_sc_kernel(n_features, batch, cpw)(
        inputsT, vals_p, rows_p, cols_p)

    f_blk_n = n_features // F_BLK
    z = pl.pallas_call(
        _epilogue_kernel,
        grid=(f_blk_n,),
        in_specs=[
            pl.BlockSpec((N_CORES, F_BLK, batch), lambda i: (0, i, 0)),
            pl.BlockSpec((1, F_BLK), lambda i: (0, i)),
            pl.BlockSpec((batch, batch), lambda i: (0, 0)),
        ],
        out_specs=pl.BlockSpec((batch, F_BLK), lambda i: (0, i)),
        out_shape=jax.ShapeDtypeStruct((batch, n_features), jnp.float32),
    )(parts, bias.reshape(1, n_features), eye)

    return z
